# baseline (speedup 1.0000x reference)
import dataclasses
import functools

import jax
import jax.numpy as jnp
from jax import lax
from jax.experimental import pallas as pl
from jax.experimental.pallas import tpu as pltpu
from jax.experimental.pallas import tpu_sc as plsc

B = 16384
D = 128
L = 16
NC = 2
NS = 16
NW = NC * NS
BPW = B // NW
C = 32


def _tree_sum(vs):
    while len(vs) > 1:
        vs = [a + b for a, b in zip(vs[::2], vs[1::2])]
    return vs[0]


def _sc_body(uid_hbm, mid_hbm, ut_hbm, mt_hbm, out_hbm,
             uid_v, mid_v, u0, u1, m0, m1, acc_v, o_v, *sems):
    wid = lax.axis_index("s") * NC + lax.axis_index("c")
    base = wid * BPW
    lane = lax.iota(jnp.int32, L)
    ubuf = (u0, u1)
    mbuf = (m0, m1)

    cu0 = pltpu.async_copy(uid_hbm.at[pl.ds(base, C)], uid_v.at[pl.ds(0, C)],
                           sems[0])
    cm0 = pltpu.async_copy(mid_hbm.at[pl.ds(base, C)], mid_v.at[pl.ds(0, C)],
                           sems[1])
    cu0.wait()
    cm0.wait()

    def issue(c0, b):
        pltpu.async_copy(ut_hbm.at[uid_v.at[pl.ds(c0, C)]], ubuf[b], sems[2 * b])
        pltpu.async_copy(mt_hbm.at[mid_v.at[pl.ds(c0, C)]], mbuf[b], sems[2 * b + 1])

    def wait(b):
        pltpu.make_async_copy(ut_hbm.at[uid_v.at[pl.ds(0, C)]], ubuf[b],
                              sems[2 * b]).wait()
        pltpu.make_async_copy(mt_hbm.at[mid_v.at[pl.ds(0, C)]], mbuf[b],
                              sems[2 * b + 1]).wait()

    def compute(c0, b):
        u_v, m_v = ubuf[b], mbuf[b]

        @pl.loop(0, C, step=L)
        def _(t0):
            @pl.loop(0, L, step=4)
            def _(jj):
                for j8 in range(4):
                    r = t0 + jj + j8
                    ps = [u_v[r, pl.ds(g * L, L)] * m_v[r, pl.ds(g * L, L)]
                          for g in range(D // L)]
                    acc_v[jj + j8, pl.ds(0, L)] = _tree_sum(ps)
            rows = [plsc.load_gather(acc_v, [lane, jnp.full((L,), j, jnp.int32)])
                    for j in range(L)]
            dv = _tree_sum(rows)
            o_v[pl.ds(c0 + t0, L)] = 10.0 / (1.0 + jnp.exp(-dv))

    issue(0, 0)
    pltpu.sync_copy(uid_hbm.at[pl.ds(base + C, BPW - C)], uid_v.at[pl.ds(C, BPW - C)])
    pltpu.sync_copy(mid_hbm.at[pl.ds(base + C, BPW - C)], mid_v.at[pl.ds(C, BPW - C)])

    @pl.loop(0, BPW, step=2 * C)
    def _(c0):
        issue(c0 + C, 1)
        wait(0)
        compute(c0, 0)

        @pl.when(c0 + 2 * C < BPW)
        def _():
            issue(c0 + 2 * C, 0)

        wait(1)
        compute(c0 + C, 1)

    pltpu.sync_copy(o_v, out_hbm.at[pl.ds(base, BPW)])


def kernel(user_id, movie_id, user_table, movie_table):
    mesh = plsc.VectorSubcoreMesh(core_axis_name="c", subcore_axis_name="s")
    cp = pltpu.CompilerParams()
    if "needs_layout_passes" in pltpu.CompilerParams.__dataclass_fields__:
        cp = dataclasses.replace(cp, needs_layout_passes=False)
    sc_k = functools.partial(
        pl.kernel,
        out_type=jax.ShapeDtypeStruct((B,), jnp.float32),
        mesh=mesh,
        compiler_params=cp,
        scratch_types=[
            pltpu.VMEM((BPW,), jnp.int32),
            pltpu.VMEM((BPW,), jnp.int32),
            pltpu.VMEM((C, D), jnp.float32),
            pltpu.VMEM((C, D), jnp.float32),
            pltpu.VMEM((C, D), jnp.float32),
            pltpu.VMEM((C, D), jnp.float32),
            pltpu.VMEM((L, L), jnp.float32),
            pltpu.VMEM((BPW,), jnp.float32),
            pltpu.SemaphoreType.DMA,
            pltpu.SemaphoreType.DMA,
            pltpu.SemaphoreType.DMA,
            pltpu.SemaphoreType.DMA,
        ],
    )(_sc_body)
    return sc_k(user_id, movie_id, user_table, movie_table)


# speedup vs baseline: 1.0282x; 1.0282x over previous
import dataclasses
import functools

import jax
import jax.numpy as jnp
from jax import lax
from jax.experimental import pallas as pl
from jax.experimental.pallas import tpu as pltpu
from jax.experimental.pallas import tpu_sc as plsc

B = 16384
D = 128
L = 16
NC = 2
NS = 16
NW = NC * NS
BPW = B // NW
C = 64


def _tree_sum(vs):
    while len(vs) > 1:
        vs = [a + b for a, b in zip(vs[::2], vs[1::2])]
    return vs[0]


def _sc_body(uid_hbm, mid_hbm, ut_hbm, mt_hbm, out_hbm,
             uid_v, mid_v, u0, u1, m0, m1, acc_v, o_v, *sems):
    wid = lax.axis_index("s") * NC + lax.axis_index("c")
    base = wid * BPW
    lane = lax.iota(jnp.int32, L)
    ubuf = (u0, u1)
    mbuf = (m0, m1)

    cu0 = pltpu.async_copy(uid_hbm.at[pl.ds(base, C)], uid_v.at[pl.ds(0, C)],
                           sems[0])
    cm0 = pltpu.async_copy(mid_hbm.at[pl.ds(base, C)], mid_v.at[pl.ds(0, C)],
                           sems[1])
    cu0.wait()
    cm0.wait()

    def issue(c0, b):
        pltpu.async_copy(ut_hbm.at[uid_v.at[pl.ds(c0, C)]], ubuf[b], sems[2 * b])
        pltpu.async_copy(mt_hbm.at[mid_v.at[pl.ds(c0, C)]], mbuf[b], sems[2 * b + 1])

    def wait(b):
        pltpu.make_async_copy(ut_hbm.at[uid_v.at[pl.ds(0, C)]], ubuf[b],
                              sems[2 * b]).wait()
        pltpu.make_async_copy(mt_hbm.at[mid_v.at[pl.ds(0, C)]], mbuf[b],
                              sems[2 * b + 1]).wait()

    def compute(c0, b):
        u_v, m_v = ubuf[b], mbuf[b]

        @pl.loop(0, C, step=L)
        def _(t0):
            @pl.loop(0, L, step=4)
            def _(jj):
                for j8 in range(4):
                    r = t0 + jj + j8
                    ps = [u_v[r, pl.ds(g * L, L)] * m_v[r, pl.ds(g * L, L)]
                          for g in range(D // L)]
                    acc_v[jj + j8, pl.ds(0, L)] = _tree_sum(ps)
            rows = [plsc.load_gather(acc_v, [lane, jnp.full((L,), j, jnp.int32)])
                    for j in range(L)]
            dv = _tree_sum(rows)
            o_v[pl.ds(c0 + t0, L)] = 10.0 / (1.0 + jnp.exp(-dv))

    issue(0, 0)
    pltpu.sync_copy(uid_hbm.at[pl.ds(base + C, BPW - C)], uid_v.at[pl.ds(C, BPW - C)])
    pltpu.sync_copy(mid_hbm.at[pl.ds(base + C, BPW - C)], mid_v.at[pl.ds(C, BPW - C)])

    @pl.loop(0, BPW, step=2 * C)
    def _(c0):
        issue(c0 + C, 1)
        wait(0)
        compute(c0, 0)

        @pl.when(c0 + 2 * C < BPW)
        def _():
            issue(c0 + 2 * C, 0)

        wait(1)
        compute(c0 + C, 1)

    pltpu.sync_copy(o_v, out_hbm.at[pl.ds(base, BPW)])


def kernel(user_id, movie_id, user_table, movie_table):
    mesh = plsc.VectorSubcoreMesh(core_axis_name="c", subcore_axis_name="s")
    cp = pltpu.CompilerParams()
    if "needs_layout_passes" in pltpu.CompilerParams.__dataclass_fields__:
        cp = dataclasses.replace(cp, needs_layout_passes=False)
    sc_k = functools.partial(
        pl.kernel,
        out_type=jax.ShapeDtypeStruct((B,), jnp.float32),
        mesh=mesh,
        compiler_params=cp,
        scratch_types=[
            pltpu.VMEM((BPW,), jnp.int32),
            pltpu.VMEM((BPW,), jnp.int32),
            pltpu.VMEM((C, D), jnp.float32),
            pltpu.VMEM((C, D), jnp.float32),
            pltpu.VMEM((C, D), jnp.float32),
            pltpu.VMEM((C, D), jnp.float32),
            pltpu.VMEM((L, L), jnp.float32),
            pltpu.VMEM((BPW,), jnp.float32),
            pltpu.SemaphoreType.DMA,
            pltpu.SemaphoreType.DMA,
            pltpu.SemaphoreType.DMA,
            pltpu.SemaphoreType.DMA,
        ],
    )(_sc_body)
    return sc_k(user_id, movie_id, user_table, movie_table)
